# baseline (speedup 1.0000x reference)
import functools

import jax
import jax.numpy as jnp
from jax.experimental import pallas as pl

N = 10000
E = 320000
H = 128
KGE = 128
G = 64
N_ITER = 2


def _matmul_bias_prelu_kern(x_ref, w_ref, b_ref, a_ref, o_ref):
    t = jnp.dot(x_ref[...], w_ref[...], preferred_element_type=jnp.float32)
    t = t + b_ref[...]
    a = a_ref[0]
    o_ref[...] = jnp.maximum(t, 0.0) + a * jnp.minimum(t, 0.0)


@jax.jit
def _matmul_bias_prelu(x, w_t, b, a):
    BLK = 2000
    grid = (x.shape[0] // BLK,)
    return pl.pallas_call(
        _matmul_bias_prelu_kern,
        grid=grid,
        in_specs=[
            pl.BlockSpec((BLK, H), lambda i: (i, 0)),
            pl.BlockSpec((H, H), lambda i: (0, 0)),
            pl.BlockSpec((H,), lambda i: (0,)),
            pl.BlockSpec((1,), lambda i: (0,)),
        ],
        out_specs=pl.BlockSpec((BLK, H), lambda i: (i, 0)),
        out_shape=jax.ShapeDtypeStruct((x.shape[0], H), jnp.float32),
    )(x, w_t, b, a.reshape(1))


def _scatter_mean(vals, idx, size):
    s = jax.ops.segment_sum(vals, idx, num_segments=size)
    cnt = jax.ops.segment_sum(jnp.ones((vals.shape[0], 1), vals.dtype), idx,
                              num_segments=size)
    return s / jnp.clip(cnt, 1.0)


def _prelu(v, a):
    return jnp.maximum(v, 0.0) + a * jnp.minimum(v, 0.0)


def _batchnorm(v, gamma, beta):
    m = jnp.mean(v, axis=0)
    var = jnp.mean((v - m) ** 2, axis=0)
    return (v - m) / jnp.sqrt(var + 1e-5) * gamma + beta


def kernel(x, edge_index, edge_attr, batch, line_graph_edge_index,
           W_e, b_e, a_e, gamma_bn, beta_bn,
           W_ih, W_hh, b_ih, b_hh,
           W_a1, b_a1, a_a, W_a2, b_a2, W_r, b_r):
    src = edge_index[0]
    dst = edge_index[1]
    lsrc = line_graph_edge_index[0]
    ldst = line_graph_edge_index[1]
    hidden = x
    xc = x
    for _ in range(N_ITER):
        proj_src = xc[src]
        proj_dst = xc[dst]
        fused = edge_attr + (proj_src + proj_dst) / 2.0
        messages = fused[lsrc]
        agg = _scatter_mean(messages, ldst, E)
        t = _matmul_bias_prelu(agg, W_e.T, b_e, a_e)
        t = _batchnorm(t, gamma_bn, beta_bn)
        fused = fused + t
        node_updates = _scatter_mean(fused, dst, N)
        gi = node_updates @ W_ih.T + b_ih
        gh = hidden @ W_hh.T + b_hh
        r = jax.nn.sigmoid(gi[:, :H] + gh[:, :H])
        z = jax.nn.sigmoid(gi[:, H:2 * H] + gh[:, H:2 * H])
        n = jnp.tanh(gi[:, 2 * H:] + r * gh[:, 2 * H:])
        hidden = (1.0 - z) * n + z * hidden
        xc = hidden
    graph_repr = _scatter_mean(xc, batch, G)
    a = _prelu(graph_repr @ W_a1.T + b_a1, a_a)
    attn = jax.nn.sigmoid(a @ W_a2.T + b_a2)
    attended = xc * attn[batch]
    pooled = jax.ops.segment_sum(attended, batch, num_segments=G)
    graph_emb = pooled @ W_r.T + b_r
    return (xc, graph_emb)


# speedup vs baseline: 1.0192x; 1.0192x over previous
import functools

import jax
import jax.numpy as jnp
from jax import lax
from jax.experimental import pallas as pl
from jax.experimental.pallas import tpu as pltpu
from jax.experimental.pallas import tpu_sc as plsc

N = 10000
E = 320000
H = 128
KGE = 128
G = 64
N_ITER = 2

NC = 2
NS = 16
NW = NC * NS
L = 16

BLK = 2000


def _stats_kern(agg_ref, pids_ref, P_ref, cnt_ref, w_ref, b_ref, a_ref,
                out_ref, acc_ref):
    i = pl.program_id(0)
    b0 = i * BLK
    agg = agg_ref[...]
    pids = pids_ref[...][:, L - 1]
    rows = b0 + lax.broadcasted_iota(jnp.int32, (BLK, NW), 0)
    mfix = (rows == pids[None, :]).astype(jnp.float32)
    agg = agg + jnp.dot(mfix, P_ref[...], preferred_element_type=jnp.float32)
    cnt = cnt_ref[...]
    aggm = jnp.where(cnt > 0, agg / jnp.clip(cnt, 1.0), 0.0)
    t = jnp.dot(aggm, w_ref[...], preferred_element_type=jnp.float32) + b_ref[...]
    t = jnp.maximum(t, 0.0) + a_ref[0] * jnp.minimum(t, 0.0)

    @pl.when(i == 0)
    def _():
        acc_ref[...] = jnp.zeros_like(acc_ref)

    acc_ref[0:1, :] += jnp.sum(t, axis=0, keepdims=True)
    acc_ref[1:2, :] += jnp.sum(t * t, axis=0, keepdims=True)

    @pl.when(i == pl.num_programs(0) - 1)
    def _():
        out_ref[...] = acc_ref[...]


def _fuse2_kern(agg_ref, pids_ref, P_ref, cnt_ref, fused_ref, stats_ref,
                w_ref, b_ref, a_ref, g_ref, be_ref, out_ref):
    i = pl.program_id(0)
    b0 = i * BLK
    agg = agg_ref[...]
    pids = pids_ref[...][:, L - 1]
    rows = b0 + lax.broadcasted_iota(jnp.int32, (BLK, NW), 0)
    mfix = (rows == pids[None, :]).astype(jnp.float32)
    agg = agg + jnp.dot(mfix, P_ref[...], preferred_element_type=jnp.float32)
    cnt = cnt_ref[...]
    aggm = jnp.where(cnt > 0, agg / jnp.clip(cnt, 1.0), 0.0)
    t = jnp.dot(aggm, w_ref[...], preferred_element_type=jnp.float32) + b_ref[...]
    t = jnp.maximum(t, 0.0) + a_ref[0] * jnp.minimum(t, 0.0)
    mean = stats_ref[0:1, :] / E
    var = stats_ref[1:2, :] / E - mean * mean
    rstd = lax.rsqrt(var + 1e-5)
    out_ref[...] = fused_ref[...] + (t - mean) * rstd * g_ref[...] + be_ref[...]


def _gru_kern(nu2_ref, cntd_ref, h_ref, wih_ref, whh_ref, bih_ref, bhh_ref,
              out_ref):
    nu = (nu2_ref[0] + nu2_ref[1]) / jnp.clip(cntd_ref[...], 1.0)
    gi = jnp.dot(nu, wih_ref[...], preferred_element_type=jnp.float32) + bih_ref[...]
    h = h_ref[...]
    gh = jnp.dot(h, whh_ref[...], preferred_element_type=jnp.float32) + bhh_ref[...]
    r = jax.nn.sigmoid(gi[:, :H] + gh[:, :H])
    z = jax.nn.sigmoid(gi[:, H:2 * H] + gh[:, H:2 * H])
    n = jnp.tanh(gi[:, 2 * H:] + r * gh[:, 2 * H:])
    out_ref[...] = (1.0 - z) * n + z * h


def _readout_kern(xc_ref, batch_ref, wa1_ref, ba1_ref, aa_ref, wa2_ref,
                  ba2_ref, wr_ref, br_ref, out_ref):
    xc = xc_ref[...]
    b = batch_ref[...]
    onehot = (b == lax.broadcasted_iota(jnp.int32, (N, G), 1)).astype(jnp.float32)
    cnt = jnp.sum(onehot, axis=0, keepdims=True)
    ssum = lax.dot_general(onehot, xc, (((0,), (0,)), ((), ())),
                           preferred_element_type=jnp.float32)
    grep = ssum / jnp.clip(cnt.T, 1.0)
    a = jnp.dot(grep, wa1_ref[...], preferred_element_type=jnp.float32) + ba1_ref[...]
    a = jnp.maximum(a, 0.0) + aa_ref[0] * jnp.minimum(a, 0.0)
    logits = jnp.sum(a * wa2_ref[...], axis=1, keepdims=True) + ba2_ref[0, 0]
    attn = jax.nn.sigmoid(logits)
    node_attn = jnp.dot(onehot, attn, preferred_element_type=jnp.float32)
    attended = xc * node_attn
    pooled = lax.dot_general(onehot, attended, (((0,), (0,)), ((), ())),
                             preferred_element_type=jnp.float32)
    out_ref[...] = jnp.dot(pooled, wr_ref[...], preferred_element_type=jnp.float32) + br_ref[...]


def _tc_stats(agg, pids, P, cnt, w_t, b, a):
    grid = (E // BLK,)
    return pl.pallas_call(
        _stats_kern,
        grid=grid,
        in_specs=[
            pl.BlockSpec((BLK, H), lambda i: (i, 0)),
            pl.BlockSpec((NW, L), lambda i: (0, 0)),
            pl.BlockSpec((NW, H), lambda i: (0, 0)),
            pl.BlockSpec((BLK, 1), lambda i: (i, 0)),
            pl.BlockSpec((H, H), lambda i: (0, 0)),
            pl.BlockSpec((1, H), lambda i: (0, 0)),
            pl.BlockSpec((1,), lambda i: (0,)),
        ],
        out_specs=pl.BlockSpec((8, H), lambda i: (0, 0)),
        out_shape=jax.ShapeDtypeStruct((8, H), jnp.float32),
        scratch_shapes=[pltpu.VMEM((8, H), jnp.float32)],
    )(agg, pids, P, cnt, w_t, b, a)


def _tc_fuse2(agg, pids, P, cnt, fused, stats, w_t, b, a, gamma, beta):
    grid = (E // BLK,)
    return pl.pallas_call(
        _fuse2_kern,
        grid=grid,
        in_specs=[
            pl.BlockSpec((BLK, H), lambda i: (i, 0)),
            pl.BlockSpec((NW, L), lambda i: (0, 0)),
            pl.BlockSpec((NW, H), lambda i: (0, 0)),
            pl.BlockSpec((BLK, 1), lambda i: (i, 0)),
            pl.BlockSpec((BLK, H), lambda i: (i, 0)),
            pl.BlockSpec((8, H), lambda i: (0, 0)),
            pl.BlockSpec((H, H), lambda i: (0, 0)),
            pl.BlockSpec((1, H), lambda i: (0, 0)),
            pl.BlockSpec((1,), lambda i: (0,)),
            pl.BlockSpec((1, H), lambda i: (0, 0)),
            pl.BlockSpec((1, H), lambda i: (0, 0)),
        ],
        out_specs=pl.BlockSpec((BLK, H), lambda i: (i, 0)),
        out_shape=jax.ShapeDtypeStruct((E, H), jnp.float32),
    )(agg, pids, P, cnt, fused, stats, w_t, b, a, gamma, beta)


def _tc_gru(nu2, cnt_d, hidden, wih_t, whh_t, bih, bhh):
    NBLK = 2000
    grid = (N // NBLK,)
    return pl.pallas_call(
        _gru_kern,
        grid=grid,
        in_specs=[
            pl.BlockSpec((2, NBLK, H), lambda i: (0, i, 0)),
            pl.BlockSpec((NBLK, 1), lambda i: (i, 0)),
            pl.BlockSpec((NBLK, H), lambda i: (i, 0)),
            pl.BlockSpec((H, 3 * H), lambda i: (0, 0)),
            pl.BlockSpec((H, 3 * H), lambda i: (0, 0)),
            pl.BlockSpec((1, 3 * H), lambda i: (0, 0)),
            pl.BlockSpec((1, 3 * H), lambda i: (0, 0)),
        ],
        out_specs=pl.BlockSpec((NBLK, H), lambda i: (i, 0)),
        out_shape=jax.ShapeDtypeStruct((N, H), jnp.float32),
    )(nu2, cnt_d, hidden, wih_t, whh_t, bih, bhh)


def _tc_readout(xc, batch2, wa1_t, ba1, aa, wa2, ba2, wr_t, br):
    return pl.pallas_call(
        _readout_kern,
        grid=(1,),
        in_specs=[
            pl.BlockSpec((N, H), lambda i: (0, 0)),
            pl.BlockSpec((N, 1), lambda i: (0, 0)),
            pl.BlockSpec((H, H // 2), lambda i: (0, 0)),
            pl.BlockSpec((1, H // 2), lambda i: (0, 0)),
            pl.BlockSpec((1,), lambda i: (0,)),
            pl.BlockSpec((1, H // 2), lambda i: (0, 0)),
            pl.BlockSpec((1, 1), lambda i: (0, 0)),
            pl.BlockSpec((H, KGE), lambda i: (0, 0)),
            pl.BlockSpec((1, KGE), lambda i: (0, 0)),
        ],
        out_specs=pl.BlockSpec((G, KGE), lambda i: (0, 0)),
        out_shape=jax.ShapeDtypeStruct((G, KGE), jnp.float32),
    )(xc, batch2, wa1_t, ba1, aa, wa2, ba2, wr_t, br)


def kernel(x, edge_index, edge_attr, batch, line_graph_edge_index,
           W_e, b_e, a_e, gamma_bn, beta_bn,
           W_ih, W_hh, b_ih, b_hh,
           W_a1, b_a1, a_a, W_a2, b_a2, W_r, b_r):
    src = edge_index[0]
    dst = edge_index[1]
    lsrc = line_graph_edge_index[0]
    ldst = line_graph_edge_index[1]

    perm = jnp.argsort(ldst)
    gidx = lsrc[perm]
    cnt_l = jax.ops.segment_sum(jnp.ones((E, 1), jnp.float32), ldst,
                                num_segments=E)
    cnt_d = jax.ops.segment_sum(jnp.ones((E, 1), jnp.float32), dst,
                                num_segments=N)

    w_e_t = W_e.T
    b_e2 = b_e.reshape(1, H)
    gamma2 = gamma_bn.reshape(1, H)
    beta2 = beta_bn.reshape(1, H)
    wih_t = W_ih.T
    whh_t = W_hh.T
    bih2 = b_ih.reshape(1, 3 * H)
    bhh2 = b_hh.reshape(1, 3 * H)
    a_e1 = a_e.reshape(1)
    a_a1 = a_a.reshape(1)
    batch2 = batch.reshape(N, 1)

    pids0 = jnp.full((NW, L), -1, jnp.int32)
    P0 = jnp.zeros((NW, H), jnp.float32)

    hidden = x
    xc = x
    for _ in range(N_ITER):
        fused = edge_attr + (xc[src] + xc[dst]) / 2.0
        agg = jax.ops.segment_sum(fused[gidx],
                                  jnp.sort(ldst), num_segments=E)
        stats = _tc_stats(agg, pids0, P0, cnt_l, w_e_t, b_e2, a_e1)
        fused2 = _tc_fuse2(agg, pids0, P0, cnt_l, fused, stats, w_e_t, b_e2,
                           a_e1, gamma2, beta2)
        nu0 = jax.ops.segment_sum(fused2, dst, num_segments=N)
        nu2 = jnp.stack([nu0, jnp.zeros_like(nu0)])
        hidden = _tc_gru(nu2, cnt_d, hidden, wih_t, whh_t, bih2, bhh2)
        xc = hidden

    graph_emb = _tc_readout(xc, batch2, W_a1.T, b_a1.reshape(1, H // 2), a_a1,
                            W_a2.reshape(1, H // 2), b_a2.reshape(1, 1),
                            W_r.T, b_r.reshape(1, KGE))
    return (xc, graph_emb)


# speedup vs baseline: 1.3621x; 1.3364x over previous
import functools

import jax
import jax.numpy as jnp
from jax import lax
from jax.experimental import pallas as pl
from jax.experimental.pallas import tpu as pltpu
from jax.experimental.pallas import tpu_sc as plsc

N = 10000
E = 320000
H = 128
KGE = 128
G = 64
N_ITER = 2

NC = 2
NS = 16
NW = NC * NS
L = 16

BLK = 2000


def _stats_kern(agg_ref, pids_ref, P_ref, cnt_ref, w_ref, b_ref, a_ref,
                out_ref, acc_ref):
    i = pl.program_id(0)
    b0 = i * BLK
    agg = agg_ref[...]
    pids = pids_ref[...][:, L - 1]
    rows = b0 + lax.broadcasted_iota(jnp.int32, (BLK, NW), 0)
    mfix = (rows == pids[None, :]).astype(jnp.float32)
    agg = agg + jnp.dot(mfix, P_ref[...], preferred_element_type=jnp.float32)
    cnt = cnt_ref[...]
    aggm = jnp.where(cnt > 0, agg / jnp.clip(cnt, 1.0), 0.0)
    t = jnp.dot(aggm, w_ref[...], preferred_element_type=jnp.float32) + b_ref[...]
    t = jnp.maximum(t, 0.0) + a_ref[0] * jnp.minimum(t, 0.0)

    @pl.when(i == 0)
    def _():
        acc_ref[...] = jnp.zeros_like(acc_ref)

    acc_ref[0:1, :] += jnp.sum(t, axis=0, keepdims=True)
    acc_ref[1:2, :] += jnp.sum(t * t, axis=0, keepdims=True)

    @pl.when(i == pl.num_programs(0) - 1)
    def _():
        out_ref[...] = acc_ref[...]


def _fuse2_kern(agg_ref, pids_ref, P_ref, cnt_ref, fused_ref, stats_ref,
                w_ref, b_ref, a_ref, g_ref, be_ref, out_ref):
    i = pl.program_id(0)
    b0 = i * BLK
    agg = agg_ref[...]
    pids = pids_ref[...][:, L - 1]
    rows = b0 + lax.broadcasted_iota(jnp.int32, (BLK, NW), 0)
    mfix = (rows == pids[None, :]).astype(jnp.float32)
    agg = agg + jnp.dot(mfix, P_ref[...], preferred_element_type=jnp.float32)
    cnt = cnt_ref[...]
    aggm = jnp.where(cnt > 0, agg / jnp.clip(cnt, 1.0), 0.0)
    t = jnp.dot(aggm, w_ref[...], preferred_element_type=jnp.float32) + b_ref[...]
    t = jnp.maximum(t, 0.0) + a_ref[0] * jnp.minimum(t, 0.0)
    mean = stats_ref[0:1, :] / E
    var = stats_ref[1:2, :] / E - mean * mean
    rstd = lax.rsqrt(var + 1e-5)
    out_ref[...] = fused_ref[...] + (t - mean) * rstd * g_ref[...] + be_ref[...]


def _gru_kern(nu2_ref, cntd_ref, h_ref, wih_ref, whh_ref, bih_ref, bhh_ref,
              out_ref):
    nu = (nu2_ref[0] + nu2_ref[1]) / jnp.clip(cntd_ref[...], 1.0)
    gi = jnp.dot(nu, wih_ref[...], preferred_element_type=jnp.float32) + bih_ref[...]
    h = h_ref[...]
    gh = jnp.dot(h, whh_ref[...], preferred_element_type=jnp.float32) + bhh_ref[...]
    r = jax.nn.sigmoid(gi[:, :H] + gh[:, :H])
    z = jax.nn.sigmoid(gi[:, H:2 * H] + gh[:, H:2 * H])
    n = jnp.tanh(gi[:, 2 * H:] + r * gh[:, 2 * H:])
    out_ref[...] = (1.0 - z) * n + z * h


def _readout_kern(xc_ref, batch_ref, wa1_ref, ba1_ref, aa_ref, wa2_ref,
                  ba2_ref, wr_ref, br_ref, out_ref):
    xc = xc_ref[...]
    b = batch_ref[...]
    onehot = (b == lax.broadcasted_iota(jnp.int32, (N, G), 1)).astype(jnp.float32)
    cnt = jnp.sum(onehot, axis=0, keepdims=True)
    ssum = lax.dot_general(onehot, xc, (((0,), (0,)), ((), ())),
                           preferred_element_type=jnp.float32)
    grep = ssum / jnp.clip(cnt.T, 1.0)
    a = jnp.dot(grep, wa1_ref[...], preferred_element_type=jnp.float32) + ba1_ref[...]
    a = jnp.maximum(a, 0.0) + aa_ref[0] * jnp.minimum(a, 0.0)
    logits = jnp.sum(a * wa2_ref[...], axis=1, keepdims=True) + ba2_ref[0, 0]
    attn = jax.nn.sigmoid(logits)
    node_attn = jnp.dot(onehot, attn, preferred_element_type=jnp.float32)
    attended = xc * node_attn
    pooled = lax.dot_general(onehot, attended, (((0,), (0,)), ((), ())),
                             preferred_element_type=jnp.float32)
    out_ref[...] = jnp.dot(pooled, wr_ref[...], preferred_element_type=jnp.float32) + br_ref[...]


def _tc_stats(agg, pids, P, cnt, w_t, b, a):
    grid = (E // BLK,)
    return pl.pallas_call(
        _stats_kern,
        grid=grid,
        in_specs=[
            pl.BlockSpec((BLK, H), lambda i: (i, 0)),
            pl.BlockSpec((NW, L), lambda i: (0, 0)),
            pl.BlockSpec((NW, H), lambda i: (0, 0)),
            pl.BlockSpec((BLK, 1), lambda i: (i, 0)),
            pl.BlockSpec((H, H), lambda i: (0, 0)),
            pl.BlockSpec((1, H), lambda i: (0, 0)),
            pl.BlockSpec((1,), lambda i: (0,)),
        ],
        out_specs=pl.BlockSpec((8, H), lambda i: (0, 0)),
        out_shape=jax.ShapeDtypeStruct((8, H), jnp.float32),
        scratch_shapes=[pltpu.VMEM((8, H), jnp.float32)],
    )(agg, pids, P, cnt, w_t, b, a)


def _tc_fuse2(agg, pids, P, cnt, fused, stats, w_t, b, a, gamma, beta):
    grid = (E // BLK,)
    return pl.pallas_call(
        _fuse2_kern,
        grid=grid,
        in_specs=[
            pl.BlockSpec((BLK, H), lambda i: (i, 0)),
            pl.BlockSpec((NW, L), lambda i: (0, 0)),
            pl.BlockSpec((NW, H), lambda i: (0, 0)),
            pl.BlockSpec((BLK, 1), lambda i: (i, 0)),
            pl.BlockSpec((BLK, H), lambda i: (i, 0)),
            pl.BlockSpec((8, H), lambda i: (0, 0)),
            pl.BlockSpec((H, H), lambda i: (0, 0)),
            pl.BlockSpec((1, H), lambda i: (0, 0)),
            pl.BlockSpec((1,), lambda i: (0,)),
            pl.BlockSpec((1, H), lambda i: (0, 0)),
            pl.BlockSpec((1, H), lambda i: (0, 0)),
        ],
        out_specs=pl.BlockSpec((BLK, H), lambda i: (i, 0)),
        out_shape=jax.ShapeDtypeStruct((E, H), jnp.float32),
    )(agg, pids, P, cnt, fused, stats, w_t, b, a, gamma, beta)


def _tc_gru(nu2, cnt_d, hidden, wih_t, whh_t, bih, bhh):
    NBLK = 2000
    grid = (N // NBLK,)
    return pl.pallas_call(
        _gru_kern,
        grid=grid,
        in_specs=[
            pl.BlockSpec((2, NBLK, H), lambda i: (0, i, 0)),
            pl.BlockSpec((NBLK, 1), lambda i: (i, 0)),
            pl.BlockSpec((NBLK, H), lambda i: (i, 0)),
            pl.BlockSpec((H, 3 * H), lambda i: (0, 0)),
            pl.BlockSpec((H, 3 * H), lambda i: (0, 0)),
            pl.BlockSpec((1, 3 * H), lambda i: (0, 0)),
            pl.BlockSpec((1, 3 * H), lambda i: (0, 0)),
        ],
        out_specs=pl.BlockSpec((NBLK, H), lambda i: (i, 0)),
        out_shape=jax.ShapeDtypeStruct((N, H), jnp.float32),
    )(nu2, cnt_d, hidden, wih_t, whh_t, bih, bhh)


def _tc_readout(xc, batch2, wa1_t, ba1, aa, wa2, ba2, wr_t, br):
    return pl.pallas_call(
        _readout_kern,
        grid=(1,),
        in_specs=[
            pl.BlockSpec((N, H), lambda i: (0, 0)),
            pl.BlockSpec((N, 1), lambda i: (0, 0)),
            pl.BlockSpec((H, H // 2), lambda i: (0, 0)),
            pl.BlockSpec((1, H // 2), lambda i: (0, 0)),
            pl.BlockSpec((1,), lambda i: (0,)),
            pl.BlockSpec((1, H // 2), lambda i: (0, 0)),
            pl.BlockSpec((1, 1), lambda i: (0, 0)),
            pl.BlockSpec((H, KGE), lambda i: (0, 0)),
            pl.BlockSpec((1, KGE), lambda i: (0, 0)),
        ],
        out_specs=pl.BlockSpec((G, KGE), lambda i: (0, 0)),
        out_shape=jax.ShapeDtypeStruct((G, KGE), jnp.float32),
    )(xc, batch2, wa1_t, ba1, aa, wa2, ba2, wr_t, br)


KD = 80
EPT = E // NW
N_PAD = 10240
NPT = N_PAD // NS

_MESH = plsc.VectorSubcoreMesh(core_axis_name="c", subcore_axis_name="s")


def _sc_nodeagg_body(fused2, dsti, zrows, out, idx_v, rows_v, acc_sh):
    cid = lax.axis_index("c")
    sid = lax.axis_index("s")
    wid = sid * NC + cid
    pltpu.sync_copy(zrows, acc_sh.at[pl.ds(sid * NPT, NPT)])
    plsc.subcore_barrier()
    base = wid * EPT

    def chunk(c, carry):
        k0 = base + c * KD
        pltpu.sync_copy(dsti.at[pl.ds(k0, KD)], idx_v)
        pltpu.sync_copy(fused2.at[pl.ds(k0, KD)], rows_v)
        pltpu.sync_copy(rows_v, acc_sh.at[idx_v], add=True)
        return carry

    lax.fori_loop(0, EPT // KD, chunk, 0)
    plsc.subcore_barrier()
    pltpu.sync_copy(acc_sh.at[pl.ds(sid * NPT, NPT)],
                    out.at[cid, pl.ds(sid * NPT, NPT)])


_sc_nodeagg = pl.kernel(
    _sc_nodeagg_body,
    out_type=jax.ShapeDtypeStruct((NC, N_PAD, H), jnp.float32),
    mesh=_MESH,
    scratch_types=[
        pltpu.VMEM((KD,), jnp.int32),
        pltpu.VMEM((KD, H), jnp.float32),
        pltpu.VMEM_SHARED((N_PAD, H), jnp.float32),
    ],
)


def _sc_fuse_body(x, srci, dsti, ea, fused, sidx_v, didx_v, xs_v, xd_v, ea_v,
                  out_v, sem):
    cid = lax.axis_index("c")
    sid = lax.axis_index("s")
    wid = sid * NC + cid
    base = wid * EPT

    def chunk(c, carry):
        k0 = base + c * KD
        pltpu.sync_copy(srci.at[pl.ds(k0, KD)], sidx_v)
        pltpu.sync_copy(dsti.at[pl.ds(k0, KD)], didx_v)
        pltpu.async_copy(x.at[sidx_v], xs_v, sem).wait()
        pltpu.async_copy(x.at[didx_v], xd_v, sem).wait()
        pltpu.sync_copy(ea.at[pl.ds(k0, KD)], ea_v)

        def row(i, rcarry):
            for c8 in range(8):
                sl = pl.ds(c8 * 16, 16)
                out_v[i, sl] = ea_v[i, sl] + 0.5 * xs_v[i, sl] + 0.5 * xd_v[i, sl]
            return rcarry

        lax.fori_loop(0, KD, row, 0)
        pltpu.sync_copy(out_v, fused.at[pl.ds(k0, KD)])
        return carry

    lax.fori_loop(0, EPT // KD, chunk, 0)


_sc_fuse = pl.kernel(
    _sc_fuse_body,
    out_type=jax.ShapeDtypeStruct((E, H), jnp.float32),
    mesh=_MESH,
    scratch_types=[
        pltpu.VMEM((KD,), jnp.int32),
        pltpu.VMEM((KD,), jnp.int32),
        pltpu.VMEM((KD, H), jnp.float32),
        pltpu.VMEM((KD, H), jnp.float32),
        pltpu.VMEM((KD, H), jnp.float32),
        pltpu.VMEM((KD, H), jnp.float32),
        pltpu.SemaphoreType.DMA,
    ],
)


def kernel(x, edge_index, edge_attr, batch, line_graph_edge_index,
           W_e, b_e, a_e, gamma_bn, beta_bn,
           W_ih, W_hh, b_ih, b_hh,
           W_a1, b_a1, a_a, W_a2, b_a2, W_r, b_r):
    src = edge_index[0]
    dst = edge_index[1]
    lsrc = line_graph_edge_index[0]
    ldst = line_graph_edge_index[1]

    perm = jnp.argsort(ldst)
    gidx = lsrc[perm]
    cnt_l = jax.ops.segment_sum(jnp.ones((E, 1), jnp.float32), ldst,
                                num_segments=E)
    cnt_d = jax.ops.segment_sum(jnp.ones((E, 1), jnp.float32), dst,
                                num_segments=N)

    w_e_t = W_e.T
    b_e2 = b_e.reshape(1, H)
    gamma2 = gamma_bn.reshape(1, H)
    beta2 = beta_bn.reshape(1, H)
    wih_t = W_ih.T
    whh_t = W_hh.T
    bih2 = b_ih.reshape(1, 3 * H)
    bhh2 = b_hh.reshape(1, 3 * H)
    a_e1 = a_e.reshape(1)
    a_a1 = a_a.reshape(1)
    batch2 = batch.reshape(N, 1)

    pids0 = jnp.full((NW, L), -1, jnp.int32)
    P0 = jnp.zeros((NW, H), jnp.float32)
    zrows = jnp.zeros((NPT, H), jnp.float32)

    hidden = x
    xc = x
    for _ in range(N_ITER):
        fused = _sc_fuse(xc, src, dst, edge_attr)
        agg = jax.ops.segment_sum(fused[gidx],
                                  jnp.sort(ldst), num_segments=E)
        stats = _tc_stats(agg, pids0, P0, cnt_l, w_e_t, b_e2, a_e1)
        fused2 = _tc_fuse2(agg, pids0, P0, cnt_l, fused, stats, w_e_t, b_e2,
                           a_e1, gamma2, beta2)
        nu2 = _sc_nodeagg(fused2, dst, zrows)
        hidden = _tc_gru(nu2, cnt_d, hidden, wih_t, whh_t, bih2, bhh2)
        xc = hidden

    graph_emb = _tc_readout(xc, batch2, W_a1.T, b_a1.reshape(1, H // 2), a_a1,
                            W_a2.reshape(1, H // 2), b_a2.reshape(1, 1),
                            W_r.T, b_r.reshape(1, KGE))
    return (xc, graph_emb)


# speedup vs baseline: 1.9343x; 1.4201x over previous
import functools

import jax
import jax.numpy as jnp
from jax import lax
from jax.experimental import pallas as pl
from jax.experimental.pallas import tpu as pltpu
from jax.experimental.pallas import tpu_sc as plsc

N = 10000
E = 320000
H = 128
KGE = 128
G = 64
N_ITER = 2

NC = 2
NS = 16
NW = NC * NS
L = 16

BLK = 2000


def _stats_kern(agg_ref, pids_ref, P_ref, cnt_ref, w_ref, b_ref, a_ref,
                out_ref, acc_ref):
    i = pl.program_id(0)
    b0 = i * BLK
    agg = agg_ref[...]
    pids = pids_ref[...][:, L - 1]
    rows = b0 + lax.broadcasted_iota(jnp.int32, (BLK, NW), 0)
    mfix = (rows == pids[None, :]).astype(jnp.float32)
    agg = agg + jnp.dot(mfix, P_ref[...], preferred_element_type=jnp.float32)
    cnt = cnt_ref[...]
    aggm = jnp.where(cnt > 0, agg / jnp.clip(cnt, 1.0), 0.0)
    t = jnp.dot(aggm, w_ref[...], preferred_element_type=jnp.float32) + b_ref[...]
    t = jnp.maximum(t, 0.0) + a_ref[0] * jnp.minimum(t, 0.0)

    @pl.when(i == 0)
    def _():
        acc_ref[...] = jnp.zeros_like(acc_ref)

    acc_ref[0:1, :] += jnp.sum(t, axis=0, keepdims=True)
    acc_ref[1:2, :] += jnp.sum(t * t, axis=0, keepdims=True)

    @pl.when(i == pl.num_programs(0) - 1)
    def _():
        out_ref[...] = acc_ref[...]


def _fuse2_kern(agg_ref, pids_ref, P_ref, cnt_ref, fused_ref, stats_ref,
                w_ref, b_ref, a_ref, g_ref, be_ref, out_ref):
    i = pl.program_id(0)
    b0 = i * BLK
    agg = agg_ref[...]
    pids = pids_ref[...][:, L - 1]
    rows = b0 + lax.broadcasted_iota(jnp.int32, (BLK, NW), 0)
    mfix = (rows == pids[None, :]).astype(jnp.float32)
    agg = agg + jnp.dot(mfix, P_ref[...], preferred_element_type=jnp.float32)
    cnt = cnt_ref[...]
    aggm = jnp.where(cnt > 0, agg / jnp.clip(cnt, 1.0), 0.0)
    t = jnp.dot(aggm, w_ref[...], preferred_element_type=jnp.float32) + b_ref[...]
    t = jnp.maximum(t, 0.0) + a_ref[0] * jnp.minimum(t, 0.0)
    mean = stats_ref[0:1, :] / E
    var = stats_ref[1:2, :] / E - mean * mean
    rstd = lax.rsqrt(var + 1e-5)
    out_ref[...] = fused_ref[...] + (t - mean) * rstd * g_ref[...] + be_ref[...]


def _gru_kern(nu2_ref, cntd_ref, h_ref, wih_ref, whh_ref, bih_ref, bhh_ref,
              out_ref):
    nu = (nu2_ref[0] + nu2_ref[1]) / jnp.clip(cntd_ref[...], 1.0)
    gi = jnp.dot(nu, wih_ref[...], preferred_element_type=jnp.float32) + bih_ref[...]
    h = h_ref[...]
    gh = jnp.dot(h, whh_ref[...], preferred_element_type=jnp.float32) + bhh_ref[...]
    r = jax.nn.sigmoid(gi[:, :H] + gh[:, :H])
    z = jax.nn.sigmoid(gi[:, H:2 * H] + gh[:, H:2 * H])
    n = jnp.tanh(gi[:, 2 * H:] + r * gh[:, 2 * H:])
    out_ref[...] = (1.0 - z) * n + z * h


def _readout_kern(xc_ref, batch_ref, wa1_ref, ba1_ref, aa_ref, wa2_ref,
                  ba2_ref, wr_ref, br_ref, out_ref):
    xc = xc_ref[...]
    b = batch_ref[...]
    onehot = (b == lax.broadcasted_iota(jnp.int32, (N, G), 1)).astype(jnp.float32)
    cnt = jnp.sum(onehot, axis=0, keepdims=True)
    ssum = lax.dot_general(onehot, xc, (((0,), (0,)), ((), ())),
                           preferred_element_type=jnp.float32)
    grep = ssum / jnp.clip(cnt.T, 1.0)
    a = jnp.dot(grep, wa1_ref[...], preferred_element_type=jnp.float32) + ba1_ref[...]
    a = jnp.maximum(a, 0.0) + aa_ref[0] * jnp.minimum(a, 0.0)
    logits = jnp.sum(a * wa2_ref[...], axis=1, keepdims=True) + ba2_ref[0, 0]
    attn = jax.nn.sigmoid(logits)
    node_attn = jnp.dot(onehot, attn, preferred_element_type=jnp.float32)
    attended = xc * node_attn
    pooled = lax.dot_general(onehot, attended, (((0,), (0,)), ((), ())),
                             preferred_element_type=jnp.float32)
    out_ref[...] = jnp.dot(pooled, wr_ref[...], preferred_element_type=jnp.float32) + br_ref[...]


def _tc_stats(agg, pids, P, cnt, w_t, b, a):
    grid = (E // BLK,)
    return pl.pallas_call(
        _stats_kern,
        grid=grid,
        in_specs=[
            pl.BlockSpec((BLK, H), lambda i: (i, 0)),
            pl.BlockSpec((NW, L), lambda i: (0, 0)),
            pl.BlockSpec((NW, H), lambda i: (0, 0)),
            pl.BlockSpec((BLK, 1), lambda i: (i, 0)),
            pl.BlockSpec((H, H), lambda i: (0, 0)),
            pl.BlockSpec((1, H), lambda i: (0, 0)),
            pl.BlockSpec((1,), lambda i: (0,)),
        ],
        out_specs=pl.BlockSpec((8, H), lambda i: (0, 0)),
        out_shape=jax.ShapeDtypeStruct((8, H), jnp.float32),
        scratch_shapes=[pltpu.VMEM((8, H), jnp.float32)],
    )(agg, pids, P, cnt, w_t, b, a)


def _tc_fuse2(agg, pids, P, cnt, fused, stats, w_t, b, a, gamma, beta):
    grid = (E // BLK,)
    return pl.pallas_call(
        _fuse2_kern,
        grid=grid,
        in_specs=[
            pl.BlockSpec((BLK, H), lambda i: (i, 0)),
            pl.BlockSpec((NW, L), lambda i: (0, 0)),
            pl.BlockSpec((NW, H), lambda i: (0, 0)),
            pl.BlockSpec((BLK, 1), lambda i: (i, 0)),
            pl.BlockSpec((BLK, H), lambda i: (i, 0)),
            pl.BlockSpec((8, H), lambda i: (0, 0)),
            pl.BlockSpec((H, H), lambda i: (0, 0)),
            pl.BlockSpec((1, H), lambda i: (0, 0)),
            pl.BlockSpec((1,), lambda i: (0,)),
            pl.BlockSpec((1, H), lambda i: (0, 0)),
            pl.BlockSpec((1, H), lambda i: (0, 0)),
        ],
        out_specs=pl.BlockSpec((BLK, H), lambda i: (i, 0)),
        out_shape=jax.ShapeDtypeStruct((E, H), jnp.float32),
    )(agg, pids, P, cnt, fused, stats, w_t, b, a, gamma, beta)


def _tc_gru(nu2, cnt_d, hidden, wih_t, whh_t, bih, bhh):
    NBLK = 2000
    grid = (N // NBLK,)
    return pl.pallas_call(
        _gru_kern,
        grid=grid,
        in_specs=[
            pl.BlockSpec((2, NBLK, H), lambda i: (0, i, 0)),
            pl.BlockSpec((NBLK, 1), lambda i: (i, 0)),
            pl.BlockSpec((NBLK, H), lambda i: (i, 0)),
            pl.BlockSpec((H, 3 * H), lambda i: (0, 0)),
            pl.BlockSpec((H, 3 * H), lambda i: (0, 0)),
            pl.BlockSpec((1, 3 * H), lambda i: (0, 0)),
            pl.BlockSpec((1, 3 * H), lambda i: (0, 0)),
        ],
        out_specs=pl.BlockSpec((NBLK, H), lambda i: (i, 0)),
        out_shape=jax.ShapeDtypeStruct((N, H), jnp.float32),
    )(nu2, cnt_d, hidden, wih_t, whh_t, bih, bhh)


def _tc_readout(xc, batch2, wa1_t, ba1, aa, wa2, ba2, wr_t, br):
    return pl.pallas_call(
        _readout_kern,
        grid=(1,),
        in_specs=[
            pl.BlockSpec((N, H), lambda i: (0, 0)),
            pl.BlockSpec((N, 1), lambda i: (0, 0)),
            pl.BlockSpec((H, H // 2), lambda i: (0, 0)),
            pl.BlockSpec((1, H // 2), lambda i: (0, 0)),
            pl.BlockSpec((1,), lambda i: (0,)),
            pl.BlockSpec((1, H // 2), lambda i: (0, 0)),
            pl.BlockSpec((1, 1), lambda i: (0, 0)),
            pl.BlockSpec((H, KGE), lambda i: (0, 0)),
            pl.BlockSpec((1, KGE), lambda i: (0, 0)),
        ],
        out_specs=pl.BlockSpec((G, KGE), lambda i: (0, 0)),
        out_shape=jax.ShapeDtypeStruct((G, KGE), jnp.float32),
    )(xc, batch2, wa1_t, ba1, aa, wa2, ba2, wr_t, br)


KD = 80
EPT = E // NW
N_PAD = 10240
NPT = N_PAD // NS

_MESH = plsc.VectorSubcoreMesh(core_axis_name="c", subcore_axis_name="s")


def _sc_nodeagg_body(fused2, dsti, zrows, out, idx_v, rows_v, acc_sh):
    cid = lax.axis_index("c")
    sid = lax.axis_index("s")
    wid = sid * NC + cid
    pltpu.sync_copy(zrows, acc_sh.at[pl.ds(sid * NPT, NPT)])
    plsc.subcore_barrier()
    base = wid * EPT

    def chunk(c, carry):
        k0 = base + c * KD
        pltpu.sync_copy(dsti.at[pl.ds(k0, KD)], idx_v)
        pltpu.sync_copy(fused2.at[pl.ds(k0, KD)], rows_v)
        pltpu.sync_copy(rows_v, acc_sh.at[idx_v], add=True)
        return carry

    lax.fori_loop(0, EPT // KD, chunk, 0)
    plsc.subcore_barrier()
    pltpu.sync_copy(acc_sh.at[pl.ds(sid * NPT, NPT)],
                    out.at[cid, pl.ds(sid * NPT, NPT)])


_sc_nodeagg = pl.kernel(
    _sc_nodeagg_body,
    out_type=jax.ShapeDtypeStruct((NC, N_PAD, H), jnp.float32),
    mesh=_MESH,
    scratch_types=[
        pltpu.VMEM((KD,), jnp.int32),
        pltpu.VMEM((KD, H), jnp.float32),
        pltpu.VMEM_SHARED((N_PAD, H), jnp.float32),
    ],
)


def _sc_fuse_body(x, srci, dsti, ea, fused, sidx_v, didx_v, xs_v, xd_v, ea_v,
                  out_v, sem):
    cid = lax.axis_index("c")
    sid = lax.axis_index("s")
    wid = sid * NC + cid
    base = wid * EPT

    def chunk(c, carry):
        k0 = base + c * KD
        pltpu.sync_copy(srci.at[pl.ds(k0, KD)], sidx_v)
        pltpu.sync_copy(dsti.at[pl.ds(k0, KD)], didx_v)
        pltpu.async_copy(x.at[sidx_v], xs_v, sem).wait()
        pltpu.async_copy(x.at[didx_v], xd_v, sem).wait()
        pltpu.sync_copy(ea.at[pl.ds(k0, KD)], ea_v)

        def row(i, rcarry):
            for c8 in range(8):
                sl = pl.ds(c8 * 16, 16)
                out_v[i, sl] = ea_v[i, sl] + 0.5 * xs_v[i, sl] + 0.5 * xd_v[i, sl]
            return rcarry

        lax.fori_loop(0, KD, row, 0)
        pltpu.sync_copy(out_v, fused.at[pl.ds(k0, KD)])
        return carry

    lax.fori_loop(0, EPT // KD, chunk, 0)


_sc_fuse = pl.kernel(
    _sc_fuse_body,
    out_type=jax.ShapeDtypeStruct((E, H), jnp.float32),
    mesh=_MESH,
    scratch_types=[
        pltpu.VMEM((KD,), jnp.int32),
        pltpu.VMEM((KD,), jnp.int32),
        pltpu.VMEM((KD, H), jnp.float32),
        pltpu.VMEM((KD, H), jnp.float32),
        pltpu.VMEM((KD, H), jnp.float32),
        pltpu.VMEM((KD, H), jnp.float32),
        pltpu.SemaphoreType.DMA,
    ],
)


def _sc_segsum_body(fused, gidxa, sidsa, aggbuf, prt, pidsf,
                    gidx_v, ids_v, oidx_v, rows_v, stage_v, prow_v, pid_v,
                    sem):
    cid = lax.axis_index("c")
    sid = lax.axis_index("s")
    wid = sid * NC + cid
    base = wid * EPT
    iota16 = lax.iota(jnp.int32, 16)

    def chunk(c, acc):
        k0 = base + c * KD
        pltpu.sync_copy(gidxa.at[pl.ds(k0, KD)], gidx_v)
        pltpu.sync_copy(sidsa.at[pl.ds(k0, KD + 16)], ids_v)
        pltpu.async_copy(fused.at[gidx_v], rows_v, sem).wait()

        def group(g, acc):
            g0 = g * 16
            ids16 = ids_v[pl.ds(g0, 16)]
            ids16n = ids_v[pl.ds(g0 + 1, 16)]
            boundary = ids16 != ids16n
            keep16 = jnp.where(boundary, 0.0, 1.0)
            dump = (E + k0 + g0) + iota16
            oidx_v[pl.ds(g0, 16)] = jnp.where(boundary, ids16, dump)
            for j in range(16):
                r = g0 + j
                acc = tuple(acc[c8] + rows_v[r, pl.ds(c8 * 16, 16)]
                            for c8 in range(8))
                for c8 in range(8):
                    stage_v[r, pl.ds(c8 * 16, 16)] = acc[c8]
                kj = jnp.broadcast_to(lax.slice(keep16, (j,), (j + 1,)), (16,))
                acc = tuple(a * kj for a in acc)
            return acc

        acc = lax.fori_loop(0, KD // 16, group, acc)
        pltpu.sync_copy(stage_v, aggbuf.at[oidx_v])
        return acc

    zero16 = jnp.zeros((16,), jnp.float32)
    acc = lax.fori_loop(0, EPT // KD, chunk, (zero16,) * 8)
    for c8 in range(8):
        prow_v[pl.ds(c8 * 16, 16)] = acc[c8]
    pltpu.sync_copy(prow_v, prt.at[pl.ds(wid * H, H)])
    pltpu.sync_copy(sidsa.at[pl.ds(base + EPT - 16, 16)], pid_v)
    pltpu.sync_copy(pid_v, pidsf.at[pl.ds(wid * L, L)])


_sc_segsum = pl.kernel(
    _sc_segsum_body,
    out_type=(jax.ShapeDtypeStruct((2 * E, H), jnp.float32),
              jax.ShapeDtypeStruct((NW * H,), jnp.float32),
              jax.ShapeDtypeStruct((NW * L,), jnp.int32)),
    mesh=_MESH,
    scratch_types=[
        pltpu.VMEM((KD,), jnp.int32),
        pltpu.VMEM((KD + 16,), jnp.int32),
        pltpu.VMEM((KD,), jnp.int32),
        pltpu.VMEM((KD, H), jnp.float32),
        pltpu.VMEM((KD, H), jnp.float32),
        pltpu.VMEM((H,), jnp.float32),
        pltpu.VMEM((L,), jnp.int32),
        pltpu.SemaphoreType.DMA,
    ],
)


def kernel(x, edge_index, edge_attr, batch, line_graph_edge_index,
           W_e, b_e, a_e, gamma_bn, beta_bn,
           W_ih, W_hh, b_ih, b_hh,
           W_a1, b_a1, a_a, W_a2, b_a2, W_r, b_r):
    src = edge_index[0]
    dst = edge_index[1]
    lsrc = line_graph_edge_index[0]
    ldst = line_graph_edge_index[1]

    perm = jnp.argsort(ldst)
    gidx = lsrc[perm]
    sids_pad = jnp.concatenate(
        [ldst[perm], jnp.full((16,), -1, jnp.int32)])
    cnt_l = jax.ops.segment_sum(jnp.ones((E, 1), jnp.float32), ldst,
                                num_segments=E)
    cnt_d = jax.ops.segment_sum(jnp.ones((E, 1), jnp.float32), dst,
                                num_segments=N)

    w_e_t = W_e.T
    b_e2 = b_e.reshape(1, H)
    gamma2 = gamma_bn.reshape(1, H)
    beta2 = beta_bn.reshape(1, H)
    wih_t = W_ih.T
    whh_t = W_hh.T
    bih2 = b_ih.reshape(1, 3 * H)
    bhh2 = b_hh.reshape(1, 3 * H)
    a_e1 = a_e.reshape(1)
    a_a1 = a_a.reshape(1)
    batch2 = batch.reshape(N, 1)

    pids0 = jnp.full((NW, L), -1, jnp.int32)
    P0 = jnp.zeros((NW, H), jnp.float32)
    zrows = jnp.zeros((NPT, H), jnp.float32)

    hidden = x
    xc = x
    for _ in range(N_ITER):
        fused = _sc_fuse(xc, src, dst, edge_attr)
        aggbuf, prt, pidsf = _sc_segsum(fused, gidx, sids_pad)
        pids = pidsf.reshape(NW, L)
        P = prt.reshape(NW, H)
        stats = _tc_stats(aggbuf, pids, P, cnt_l, w_e_t, b_e2, a_e1)
        fused2 = _tc_fuse2(aggbuf, pids, P, cnt_l, fused, stats, w_e_t, b_e2,
                           a_e1, gamma2, beta2)
        nu2 = _sc_nodeagg(fused2, dst, zrows)
        hidden = _tc_gru(nu2, cnt_d, hidden, wih_t, whh_t, bih2, bhh2)
        xc = hidden

    graph_emb = _tc_readout(xc, batch2, W_a1.T, b_a1.reshape(1, H // 2), a_a1,
                            W_a2.reshape(1, H // 2), b_a2.reshape(1, 1),
                            W_r.T, b_r.reshape(1, KGE))
    return (xc, graph_emb)


# speedup vs baseline: 2.6981x; 1.3949x over previous
import functools

import jax
import jax.numpy as jnp
from jax import lax
from jax.experimental import pallas as pl
from jax.experimental.pallas import tpu as pltpu
from jax.experimental.pallas import tpu_sc as plsc

N = 10000
E = 320000
H = 128
KGE = 128
G = 64
N_ITER = 2

NC = 2
NS = 16
NW = NC * NS
L = 16

BLK = 2000


def _stats_kern(agg_ref, pids_ref, P_ref, cnt_ref, w_ref, b_ref, a_ref,
                out_ref, acc_ref):
    i = pl.program_id(0)
    b0 = i * BLK
    agg = agg_ref[...]
    pids = pids_ref[...][:, L - 1]
    rows = b0 + lax.broadcasted_iota(jnp.int32, (BLK, NW), 0)
    mfix = (rows == pids[None, :]).astype(jnp.float32)
    agg = agg + jnp.dot(mfix, P_ref[...], preferred_element_type=jnp.float32)
    cnt = cnt_ref[...]
    aggm = jnp.where(cnt > 0, agg / jnp.clip(cnt, 1.0), 0.0)
    t = jnp.dot(aggm, w_ref[...], preferred_element_type=jnp.float32) + b_ref[...]
    t = jnp.maximum(t, 0.0) + a_ref[0] * jnp.minimum(t, 0.0)

    @pl.when(i == 0)
    def _():
        acc_ref[...] = jnp.zeros_like(acc_ref)

    acc_ref[0:1, :] += jnp.sum(t, axis=0, keepdims=True)
    acc_ref[1:2, :] += jnp.sum(t * t, axis=0, keepdims=True)

    @pl.when(i == pl.num_programs(0) - 1)
    def _():
        out_ref[...] = acc_ref[...]


def _fuse2_kern(agg_ref, pids_ref, P_ref, cnt_ref, fused_ref, stats_ref,
                w_ref, b_ref, a_ref, g_ref, be_ref, out_ref):
    i = pl.program_id(0)
    b0 = i * BLK
    agg = agg_ref[...]
    pids = pids_ref[...][:, L - 1]
    rows = b0 + lax.broadcasted_iota(jnp.int32, (BLK, NW), 0)
    mfix = (rows == pids[None, :]).astype(jnp.float32)
    agg = agg + jnp.dot(mfix, P_ref[...], preferred_element_type=jnp.float32)
    cnt = cnt_ref[...]
    aggm = jnp.where(cnt > 0, agg / jnp.clip(cnt, 1.0), 0.0)
    t = jnp.dot(aggm, w_ref[...], preferred_element_type=jnp.float32) + b_ref[...]
    t = jnp.maximum(t, 0.0) + a_ref[0] * jnp.minimum(t, 0.0)
    mean = stats_ref[0:1, :] / E
    var = stats_ref[1:2, :] / E - mean * mean
    rstd = lax.rsqrt(var + 1e-5)
    out_ref[...] = fused_ref[...] + (t - mean) * rstd * g_ref[...] + be_ref[...]


def _gru_kern(nu2_ref, cntd_ref, h_ref, wih_ref, whh_ref, bih_ref, bhh_ref,
              out_ref):
    nu = (nu2_ref[0] + nu2_ref[1]) / jnp.clip(cntd_ref[...], 1.0)
    gi = jnp.dot(nu, wih_ref[...], preferred_element_type=jnp.float32) + bih_ref[...]
    h = h_ref[...]
    gh = jnp.dot(h, whh_ref[...], preferred_element_type=jnp.float32) + bhh_ref[...]
    r = jax.nn.sigmoid(gi[:, :H] + gh[:, :H])
    z = jax.nn.sigmoid(gi[:, H:2 * H] + gh[:, H:2 * H])
    n = jnp.tanh(gi[:, 2 * H:] + r * gh[:, 2 * H:])
    out_ref[...] = (1.0 - z) * n + z * h


def _readout_kern(xc_ref, batch_ref, wa1_ref, ba1_ref, aa_ref, wa2_ref,
                  ba2_ref, wr_ref, br_ref, out_ref):
    xc = xc_ref[...]
    b = batch_ref[...]
    onehot = (b == lax.broadcasted_iota(jnp.int32, (N, G), 1)).astype(jnp.float32)
    cnt = jnp.sum(onehot, axis=0, keepdims=True)
    ssum = lax.dot_general(onehot, xc, (((0,), (0,)), ((), ())),
                           preferred_element_type=jnp.float32)
    grep = ssum / jnp.clip(cnt.T, 1.0)
    a = jnp.dot(grep, wa1_ref[...], preferred_element_type=jnp.float32) + ba1_ref[...]
    a = jnp.maximum(a, 0.0) + aa_ref[0] * jnp.minimum(a, 0.0)
    logits = jnp.sum(a * wa2_ref[...], axis=1, keepdims=True) + ba2_ref[0, 0]
    attn = jax.nn.sigmoid(logits)
    node_attn = jnp.dot(onehot, attn, preferred_element_type=jnp.float32)
    attended = xc * node_attn
    pooled = lax.dot_general(onehot, attended, (((0,), (0,)), ((), ())),
                             preferred_element_type=jnp.float32)
    out_ref[...] = jnp.dot(pooled, wr_ref[...], preferred_element_type=jnp.float32) + br_ref[...]


def _tc_stats(agg, pids, P, cnt, w_t, b, a):
    grid = (E // BLK,)
    return pl.pallas_call(
        _stats_kern,
        grid=grid,
        in_specs=[
            pl.BlockSpec((BLK, H), lambda i: (i, 0)),
            pl.BlockSpec((NW, L), lambda i: (0, 0)),
            pl.BlockSpec((NW, H), lambda i: (0, 0)),
            pl.BlockSpec((BLK, 1), lambda i: (i, 0)),
            pl.BlockSpec((H, H), lambda i: (0, 0)),
            pl.BlockSpec((1, H), lambda i: (0, 0)),
            pl.BlockSpec((1,), lambda i: (0,)),
        ],
        out_specs=pl.BlockSpec((8, H), lambda i: (0, 0)),
        out_shape=jax.ShapeDtypeStruct((8, H), jnp.float32),
        scratch_shapes=[pltpu.VMEM((8, H), jnp.float32)],
    )(agg, pids, P, cnt, w_t, b, a)


def _tc_fuse2(agg, pids, P, cnt, fused, stats, w_t, b, a, gamma, beta):
    grid = (E // BLK,)
    return pl.pallas_call(
        _fuse2_kern,
        grid=grid,
        in_specs=[
            pl.BlockSpec((BLK, H), lambda i: (i, 0)),
            pl.BlockSpec((NW, L), lambda i: (0, 0)),
            pl.BlockSpec((NW, H), lambda i: (0, 0)),
            pl.BlockSpec((BLK, 1), lambda i: (i, 0)),
            pl.BlockSpec((BLK, H), lambda i: (i, 0)),
            pl.BlockSpec((8, H), lambda i: (0, 0)),
            pl.BlockSpec((H, H), lambda i: (0, 0)),
            pl.BlockSpec((1, H), lambda i: (0, 0)),
            pl.BlockSpec((1,), lambda i: (0,)),
            pl.BlockSpec((1, H), lambda i: (0, 0)),
            pl.BlockSpec((1, H), lambda i: (0, 0)),
        ],
        out_specs=pl.BlockSpec((BLK, H), lambda i: (i, 0)),
        out_shape=jax.ShapeDtypeStruct((E, H), jnp.float32),
    )(agg, pids, P, cnt, fused, stats, w_t, b, a, gamma, beta)


def _tc_gru(nu2, cnt_d, hidden, wih_t, whh_t, bih, bhh):
    NBLK = 2000
    grid = (N // NBLK,)
    return pl.pallas_call(
        _gru_kern,
        grid=grid,
        in_specs=[
            pl.BlockSpec((2, NBLK, H), lambda i: (0, i, 0)),
            pl.BlockSpec((NBLK, 1), lambda i: (i, 0)),
            pl.BlockSpec((NBLK, H), lambda i: (i, 0)),
            pl.BlockSpec((H, 3 * H), lambda i: (0, 0)),
            pl.BlockSpec((H, 3 * H), lambda i: (0, 0)),
            pl.BlockSpec((1, 3 * H), lambda i: (0, 0)),
            pl.BlockSpec((1, 3 * H), lambda i: (0, 0)),
        ],
        out_specs=pl.BlockSpec((NBLK, H), lambda i: (i, 0)),
        out_shape=jax.ShapeDtypeStruct((N, H), jnp.float32),
    )(nu2, cnt_d, hidden, wih_t, whh_t, bih, bhh)


def _tc_readout(xc, batch2, wa1_t, ba1, aa, wa2, ba2, wr_t, br):
    return pl.pallas_call(
        _readout_kern,
        grid=(1,),
        in_specs=[
            pl.BlockSpec((N, H), lambda i: (0, 0)),
            pl.BlockSpec((N, 1), lambda i: (0, 0)),
            pl.BlockSpec((H, H // 2), lambda i: (0, 0)),
            pl.BlockSpec((1, H // 2), lambda i: (0, 0)),
            pl.BlockSpec((1,), lambda i: (0,)),
            pl.BlockSpec((1, H // 2), lambda i: (0, 0)),
            pl.BlockSpec((1, 1), lambda i: (0, 0)),
            pl.BlockSpec((H, KGE), lambda i: (0, 0)),
            pl.BlockSpec((1, KGE), lambda i: (0, 0)),
        ],
        out_specs=pl.BlockSpec((G, KGE), lambda i: (0, 0)),
        out_shape=jax.ShapeDtypeStruct((G, KGE), jnp.float32),
    )(xc, batch2, wa1_t, ba1, aa, wa2, ba2, wr_t, br)


KD = 80
EPT = E // NW
N_PAD = 10240
NPT = N_PAD // NS

_MESH = plsc.VectorSubcoreMesh(core_axis_name="c", subcore_axis_name="s")


def _sc_nodeagg_body(fused2, dsti, zrows, out, idx_v, rows_v, acc_sh):
    cid = lax.axis_index("c")
    sid = lax.axis_index("s")
    wid = sid * NC + cid
    pltpu.sync_copy(zrows, acc_sh.at[pl.ds(sid * NPT, NPT)])
    plsc.subcore_barrier()
    base = wid * EPT

    def chunk(c, carry):
        k0 = base + c * KD
        pltpu.sync_copy(dsti.at[pl.ds(k0, KD)], idx_v)
        pltpu.sync_copy(fused2.at[pl.ds(k0, KD)], rows_v)
        pltpu.sync_copy(rows_v, acc_sh.at[idx_v], add=True)
        return carry

    lax.fori_loop(0, EPT // KD, chunk, 0)
    plsc.subcore_barrier()
    pltpu.sync_copy(acc_sh.at[pl.ds(sid * NPT, NPT)],
                    out.at[cid, pl.ds(sid * NPT, NPT)])


_sc_nodeagg = pl.kernel(
    _sc_nodeagg_body,
    out_type=jax.ShapeDtypeStruct((NC, N_PAD, H), jnp.float32),
    mesh=_MESH,
    scratch_types=[
        pltpu.VMEM((KD,), jnp.int32),
        pltpu.VMEM((KD, H), jnp.float32),
        pltpu.VMEM_SHARED((N_PAD, H), jnp.float32),
    ],
)


NCH = EPT // KD
NB2 = (NCH - 1) // 2


def _sc_fuse_body(x, srci3, dsti3, ea, fused,
                  sidx_a, didx_a, xs0, xs1, xd0, xd1, ea0, ea1, out0, out1,
                  gsem0, gsem1, wsem0, wsem1):
    cid = lax.axis_index("c")
    sid = lax.axis_index("s")
    wid = sid * NC + cid
    base = wid * EPT
    pltpu.sync_copy(srci3.at[wid], sidx_a)
    pltpu.sync_copy(dsti3.at[wid], didx_a)

    def gathers(c, xs_v, xd_v, ea_v, sem):
        pltpu.async_copy(x.at[sidx_a.at[c]], xs_v, sem)
        pltpu.async_copy(x.at[didx_a.at[c]], xd_v, sem)
        pltpu.async_copy(ea.at[pl.ds(base + c * KD, KD)], ea_v, sem)

    def wait_gathers(xs_v, xd_v, ea_v, sem):
        pltpu.make_async_copy(x.at[sidx_a.at[0]], xs_v, sem).wait()
        pltpu.make_async_copy(x.at[didx_a.at[0]], xd_v, sem).wait()
        pltpu.make_async_copy(ea.at[pl.ds(base, KD)], ea_v, sem).wait()

    def compute(xs_v, xd_v, ea_v, out_v):
        def row(i, rcarry):
            for c8 in range(8):
                sl = pl.ds(c8 * 16, 16)
                out_v[i, sl] = (ea_v[i, sl] + 0.5 * xs_v[i, sl]
                                + 0.5 * xd_v[i, sl])
            return rcarry
        lax.fori_loop(0, KD, row, 0)

    gathers(0, xs0, xd0, ea0, gsem0)

    def body(t, carry):
        c0 = 2 * t
        gathers(c0 + 1, xs1, xd1, ea1, gsem1)
        wait_gathers(xs0, xd0, ea0, gsem0)
        compute(xs0, xd0, ea0, out0)
        pltpu.async_copy(out0, fused.at[pl.ds(base + c0 * KD, KD)], wsem0)
        gathers(c0 + 2, xs0, xd0, ea0, gsem0)
        wait_gathers(xs1, xd1, ea1, gsem1)
        compute(xs1, xd1, ea1, out1)
        pltpu.async_copy(out1, fused.at[pl.ds(base + (c0 + 1) * KD, KD)], wsem1)
        pltpu.make_async_copy(out0, fused.at[pl.ds(base, KD)], wsem0).wait()
        pltpu.make_async_copy(out1, fused.at[pl.ds(base, KD)], wsem1).wait()
        return carry

    lax.fori_loop(0, NB2, body, 0)
    wait_gathers(xs0, xd0, ea0, gsem0)
    compute(xs0, xd0, ea0, out0)
    pltpu.sync_copy(out0, fused.at[pl.ds(base + (NCH - 1) * KD, KD)])


_sc_fuse = pl.kernel(
    _sc_fuse_body,
    out_type=jax.ShapeDtypeStruct((E, H), jnp.float32),
    mesh=_MESH,
    scratch_types=[
        pltpu.VMEM((NCH, KD), jnp.int32),
        pltpu.VMEM((NCH, KD), jnp.int32),
        pltpu.VMEM((KD, H), jnp.float32),
        pltpu.VMEM((KD, H), jnp.float32),
        pltpu.VMEM((KD, H), jnp.float32),
        pltpu.VMEM((KD, H), jnp.float32),
        pltpu.VMEM((KD, H), jnp.float32),
        pltpu.VMEM((KD, H), jnp.float32),
        pltpu.VMEM((KD, H), jnp.float32),
        pltpu.VMEM((KD, H), jnp.float32),
        pltpu.SemaphoreType.DMA,
        pltpu.SemaphoreType.DMA,
        pltpu.SemaphoreType.DMA,
        pltpu.SemaphoreType.DMA,
    ],
)


def _sc_segsum_body(fused, gidx3, sidsa, aggbuf, prt, pidsf,
                    gidx_a, ids_a, oidx0, oidx1, rows0, rows1, stage0, stage1,
                    prow_v, pid_v, gsem0, gsem1, ssem0, ssem1):
    cid = lax.axis_index("c")
    sid = lax.axis_index("s")
    wid = sid * NC + cid
    base = wid * EPT
    iota16 = lax.iota(jnp.int32, 16)
    pltpu.sync_copy(gidx3.at[wid], gidx_a)
    pltpu.sync_copy(sidsa.at[pl.ds(base, EPT + 16)], ids_a)

    def compute(c, rows_v, stage_v, oidx_v, acc):
        coff = c * KD
        k0 = base + coff

        def group(g, acc):
            g0 = g * 16
            ids16 = ids_a[pl.ds(coff + g0, 16)]
            ids16n = ids_a[pl.ds(coff + g0 + 1, 16)]
            boundary = ids16 != ids16n
            keep16 = jnp.where(boundary, 0.0, 1.0)
            dump = (E + k0 + g0) + iota16
            oidx_v[pl.ds(g0, 16)] = jnp.where(boundary, ids16, dump)
            for j in range(16):
                r = g0 + j
                acc = tuple(acc[c8] + rows_v[r, pl.ds(c8 * 16, 16)]
                            for c8 in range(8))
                for c8 in range(8):
                    stage_v[r, pl.ds(c8 * 16, 16)] = acc[c8]
                kj = jnp.broadcast_to(lax.slice(keep16, (j,), (j + 1,)), (16,))
                acc = tuple(a * kj for a in acc)
            return acc

        return lax.fori_loop(0, KD // 16, group, acc)

    pltpu.async_copy(fused.at[gidx_a.at[0]], rows0, gsem0)

    def body(t, acc):
        c0 = 2 * t
        pltpu.async_copy(fused.at[gidx_a.at[c0 + 1]], rows1, gsem1)
        pltpu.make_async_copy(fused.at[gidx_a.at[0]], rows0, gsem0).wait()
        acc = compute(c0, rows0, stage0, oidx0, acc)
        pltpu.async_copy(stage0, aggbuf.at[oidx0], ssem0)
        pltpu.async_copy(fused.at[gidx_a.at[c0 + 2]], rows0, gsem0)
        pltpu.make_async_copy(fused.at[gidx_a.at[0]], rows1, gsem1).wait()
        acc = compute(c0 + 1, rows1, stage1, oidx1, acc)
        pltpu.async_copy(stage1, aggbuf.at[oidx1], ssem1)
        pltpu.make_async_copy(stage0, aggbuf.at[oidx0], ssem0).wait()
        pltpu.make_async_copy(stage1, aggbuf.at[oidx1], ssem1).wait()
        return acc

    zero16 = jnp.zeros((16,), jnp.float32)
    acc = lax.fori_loop(0, NB2, body, (zero16,) * 8)
    pltpu.make_async_copy(fused.at[gidx_a.at[0]], rows0, gsem0).wait()
    acc = compute(NCH - 1, rows0, stage0, oidx0, acc)
    pltpu.sync_copy(stage0, aggbuf.at[oidx0])
    for c8 in range(8):
        prow_v[pl.ds(c8 * 16, 16)] = acc[c8]
    pltpu.sync_copy(prow_v, prt.at[pl.ds(wid * H, H)])
    pltpu.sync_copy(sidsa.at[pl.ds(base + EPT - 16, 16)], pid_v)
    pltpu.sync_copy(pid_v, pidsf.at[pl.ds(wid * L, L)])


_sc_segsum = pl.kernel(
    _sc_segsum_body,
    out_type=(jax.ShapeDtypeStruct((2 * E, H), jnp.float32),
              jax.ShapeDtypeStruct((NW * H,), jnp.float32),
              jax.ShapeDtypeStruct((NW * L,), jnp.int32)),
    mesh=_MESH,
    scratch_types=[
        pltpu.VMEM((NCH, KD), jnp.int32),
        pltpu.VMEM((EPT + 16,), jnp.int32),
        pltpu.VMEM((KD,), jnp.int32),
        pltpu.VMEM((KD,), jnp.int32),
        pltpu.VMEM((KD, H), jnp.float32),
        pltpu.VMEM((KD, H), jnp.float32),
        pltpu.VMEM((KD, H), jnp.float32),
        pltpu.VMEM((KD, H), jnp.float32),
        pltpu.VMEM((H,), jnp.float32),
        pltpu.VMEM((L,), jnp.int32),
        pltpu.SemaphoreType.DMA,
        pltpu.SemaphoreType.DMA,
        pltpu.SemaphoreType.DMA,
        pltpu.SemaphoreType.DMA,
    ],
)


def kernel(x, edge_index, edge_attr, batch, line_graph_edge_index,
           W_e, b_e, a_e, gamma_bn, beta_bn,
           W_ih, W_hh, b_ih, b_hh,
           W_a1, b_a1, a_a, W_a2, b_a2, W_r, b_r):
    src = edge_index[0]
    dst = edge_index[1]
    lsrc = line_graph_edge_index[0]
    ldst = line_graph_edge_index[1]

    perm = jnp.argsort(ldst)
    gidx3 = lsrc[perm].reshape(NW, NCH, KD)
    sids_pad = jnp.concatenate(
        [ldst[perm], jnp.full((16,), -1, jnp.int32)])
    src3 = src.reshape(NW, NCH, KD)
    dst3 = dst.reshape(NW, NCH, KD)
    cnt_l = jax.ops.segment_sum(jnp.ones((E, 1), jnp.float32), ldst,
                                num_segments=E)
    cnt_d = jax.ops.segment_sum(jnp.ones((E, 1), jnp.float32), dst,
                                num_segments=N)

    w_e_t = W_e.T
    b_e2 = b_e.reshape(1, H)
    gamma2 = gamma_bn.reshape(1, H)
    beta2 = beta_bn.reshape(1, H)
    wih_t = W_ih.T
    whh_t = W_hh.T
    bih2 = b_ih.reshape(1, 3 * H)
    bhh2 = b_hh.reshape(1, 3 * H)
    a_e1 = a_e.reshape(1)
    a_a1 = a_a.reshape(1)
    batch2 = batch.reshape(N, 1)

    pids0 = jnp.full((NW, L), -1, jnp.int32)
    P0 = jnp.zeros((NW, H), jnp.float32)
    zrows = jnp.zeros((NPT, H), jnp.float32)

    hidden = x
    xc = x
    for _ in range(N_ITER):
        fused = _sc_fuse(xc, src3, dst3, edge_attr)
        aggbuf, prt, pidsf = _sc_segsum(fused, gidx3, sids_pad)
        pids = pidsf.reshape(NW, L)
        P = prt.reshape(NW, H)
        stats = _tc_stats(aggbuf, pids, P, cnt_l, w_e_t, b_e2, a_e1)
        fused2 = _tc_fuse2(aggbuf, pids, P, cnt_l, fused, stats, w_e_t, b_e2,
                           a_e1, gamma2, beta2)
        nu2 = _sc_nodeagg(fused2, dst, zrows)
        hidden = _tc_gru(nu2, cnt_d, hidden, wih_t, whh_t, bih2, bhh2)
        xc = hidden

    graph_emb = _tc_readout(xc, batch2, W_a1.T, b_a1.reshape(1, H // 2), a_a1,
                            W_a2.reshape(1, H // 2), b_a2.reshape(1, 1),
                            W_r.T, b_r.reshape(1, KGE))
    return (xc, graph_emb)


# speedup vs baseline: 2.9378x; 1.0888x over previous
import functools

import jax
import jax.numpy as jnp
from jax import lax
from jax.experimental import pallas as pl
from jax.experimental.pallas import tpu as pltpu
from jax.experimental.pallas import tpu_sc as plsc

N = 10000
E = 320000
H = 128
KGE = 128
G = 64
N_ITER = 2

NC = 2
NS = 16
NW = NC * NS
L = 16

BLK = 2000


def _stats_kern(agg_ref, pids_ref, P_ref, cnt_ref, w_ref, b_ref, a_ref,
                out_ref, acc_ref):
    i = pl.program_id(0)
    b0 = i * BLK
    agg = agg_ref[...]
    pids = pids_ref[...][:, L - 1]
    rows = b0 + lax.broadcasted_iota(jnp.int32, (BLK, NW), 0)
    mfix = (rows == pids[None, :]).astype(jnp.float32)
    agg = agg + jnp.dot(mfix, P_ref[...], preferred_element_type=jnp.float32)
    cnt = cnt_ref[...]
    aggm = jnp.where(cnt > 0, agg / jnp.clip(cnt, 1.0), 0.0)
    t = jnp.dot(aggm, w_ref[...], preferred_element_type=jnp.float32) + b_ref[...]
    t = jnp.maximum(t, 0.0) + a_ref[0] * jnp.minimum(t, 0.0)

    @pl.when(i == 0)
    def _():
        acc_ref[...] = jnp.zeros_like(acc_ref)

    acc_ref[0:1, :] += jnp.sum(t, axis=0, keepdims=True)
    acc_ref[1:2, :] += jnp.sum(t * t, axis=0, keepdims=True)

    @pl.when(i == pl.num_programs(0) - 1)
    def _():
        out_ref[...] = acc_ref[...]


def _fuse2_kern(agg_ref, pids_ref, P_ref, cnt_ref, fused_ref, stats_ref,
                w_ref, b_ref, a_ref, g_ref, be_ref, out_ref):
    i = pl.program_id(0)
    b0 = i * BLK
    agg = agg_ref[...]
    pids = pids_ref[...][:, L - 1]
    rows = b0 + lax.broadcasted_iota(jnp.int32, (BLK, NW), 0)
    mfix = (rows == pids[None, :]).astype(jnp.float32)
    agg = agg + jnp.dot(mfix, P_ref[...], preferred_element_type=jnp.float32)
    cnt = cnt_ref[...]
    aggm = jnp.where(cnt > 0, agg / jnp.clip(cnt, 1.0), 0.0)
    t = jnp.dot(aggm, w_ref[...], preferred_element_type=jnp.float32) + b_ref[...]
    t = jnp.maximum(t, 0.0) + a_ref[0] * jnp.minimum(t, 0.0)
    mean = stats_ref[0:1, :] / E
    var = stats_ref[1:2, :] / E - mean * mean
    rstd = lax.rsqrt(var + 1e-5)
    out_ref[...] = fused_ref[...] + (t - mean) * rstd * g_ref[...] + be_ref[...]


def _gru_kern(nu2_ref, cntd_ref, h_ref, wih_ref, whh_ref, bih_ref, bhh_ref,
              out_ref):
    nu = (nu2_ref[0] + nu2_ref[1]) / jnp.clip(cntd_ref[...], 1.0)
    gi = jnp.dot(nu, wih_ref[...], preferred_element_type=jnp.float32) + bih_ref[...]
    h = h_ref[...]
    gh = jnp.dot(h, whh_ref[...], preferred_element_type=jnp.float32) + bhh_ref[...]
    r = jax.nn.sigmoid(gi[:, :H] + gh[:, :H])
    z = jax.nn.sigmoid(gi[:, H:2 * H] + gh[:, H:2 * H])
    n = jnp.tanh(gi[:, 2 * H:] + r * gh[:, 2 * H:])
    out_ref[...] = (1.0 - z) * n + z * h


def _readout_kern(xc_ref, batch_ref, wa1_ref, ba1_ref, aa_ref, wa2_ref,
                  ba2_ref, wr_ref, br_ref, out_ref):
    xc = xc_ref[...]
    b = batch_ref[...]
    onehot = (b == lax.broadcasted_iota(jnp.int32, (N, G), 1)).astype(jnp.float32)
    cnt = jnp.sum(onehot, axis=0, keepdims=True)
    ssum = lax.dot_general(onehot, xc, (((0,), (0,)), ((), ())),
                           preferred_element_type=jnp.float32)
    grep = ssum / jnp.clip(cnt.T, 1.0)
    a = jnp.dot(grep, wa1_ref[...], preferred_element_type=jnp.float32) + ba1_ref[...]
    a = jnp.maximum(a, 0.0) + aa_ref[0] * jnp.minimum(a, 0.0)
    logits = jnp.sum(a * wa2_ref[...], axis=1, keepdims=True) + ba2_ref[0, 0]
    attn = jax.nn.sigmoid(logits)
    node_attn = jnp.dot(onehot, attn, preferred_element_type=jnp.float32)
    attended = xc * node_attn
    pooled = lax.dot_general(onehot, attended, (((0,), (0,)), ((), ())),
                             preferred_element_type=jnp.float32)
    out_ref[...] = jnp.dot(pooled, wr_ref[...], preferred_element_type=jnp.float32) + br_ref[...]


def _tc_stats(agg, pids, P, cnt, w_t, b, a):
    grid = (E // BLK,)
    return pl.pallas_call(
        _stats_kern,
        grid=grid,
        in_specs=[
            pl.BlockSpec((BLK, H), lambda i: (i, 0)),
            pl.BlockSpec((NW, L), lambda i: (0, 0)),
            pl.BlockSpec((NW, H), lambda i: (0, 0)),
            pl.BlockSpec((BLK, 1), lambda i: (i, 0)),
            pl.BlockSpec((H, H), lambda i: (0, 0)),
            pl.BlockSpec((1, H), lambda i: (0, 0)),
            pl.BlockSpec((1,), lambda i: (0,)),
        ],
        out_specs=pl.BlockSpec((8, H), lambda i: (0, 0)),
        out_shape=jax.ShapeDtypeStruct((8, H), jnp.float32),
        scratch_shapes=[pltpu.VMEM((8, H), jnp.float32)],
    )(agg, pids, P, cnt, w_t, b, a)


def _tc_fuse2(agg, pids, P, cnt, fused, stats, w_t, b, a, gamma, beta):
    grid = (E // BLK,)
    return pl.pallas_call(
        _fuse2_kern,
        grid=grid,
        in_specs=[
            pl.BlockSpec((BLK, H), lambda i: (i, 0)),
            pl.BlockSpec((NW, L), lambda i: (0, 0)),
            pl.BlockSpec((NW, H), lambda i: (0, 0)),
            pl.BlockSpec((BLK, 1), lambda i: (i, 0)),
            pl.BlockSpec((BLK, H), lambda i: (i, 0)),
            pl.BlockSpec((8, H), lambda i: (0, 0)),
            pl.BlockSpec((H, H), lambda i: (0, 0)),
            pl.BlockSpec((1, H), lambda i: (0, 0)),
            pl.BlockSpec((1,), lambda i: (0,)),
            pl.BlockSpec((1, H), lambda i: (0, 0)),
            pl.BlockSpec((1, H), lambda i: (0, 0)),
        ],
        out_specs=pl.BlockSpec((BLK, H), lambda i: (i, 0)),
        out_shape=jax.ShapeDtypeStruct((E, H), jnp.float32),
    )(agg, pids, P, cnt, fused, stats, w_t, b, a, gamma, beta)


def _tc_gru(nu2, cnt_d, hidden, wih_t, whh_t, bih, bhh):
    NBLK = 2000
    grid = (N // NBLK,)
    return pl.pallas_call(
        _gru_kern,
        grid=grid,
        in_specs=[
            pl.BlockSpec((2, NBLK, H), lambda i: (0, i, 0)),
            pl.BlockSpec((NBLK, 1), lambda i: (i, 0)),
            pl.BlockSpec((NBLK, H), lambda i: (i, 0)),
            pl.BlockSpec((H, 3 * H), lambda i: (0, 0)),
            pl.BlockSpec((H, 3 * H), lambda i: (0, 0)),
            pl.BlockSpec((1, 3 * H), lambda i: (0, 0)),
            pl.BlockSpec((1, 3 * H), lambda i: (0, 0)),
        ],
        out_specs=pl.BlockSpec((NBLK, H), lambda i: (i, 0)),
        out_shape=jax.ShapeDtypeStruct((N, H), jnp.float32),
    )(nu2, cnt_d, hidden, wih_t, whh_t, bih, bhh)


def _tc_readout(xc, batch2, wa1_t, ba1, aa, wa2, ba2, wr_t, br):
    return pl.pallas_call(
        _readout_kern,
        grid=(1,),
        in_specs=[
            pl.BlockSpec((N, H), lambda i: (0, 0)),
            pl.BlockSpec((N, 1), lambda i: (0, 0)),
            pl.BlockSpec((H, H // 2), lambda i: (0, 0)),
            pl.BlockSpec((1, H // 2), lambda i: (0, 0)),
            pl.BlockSpec((1,), lambda i: (0,)),
            pl.BlockSpec((1, H // 2), lambda i: (0, 0)),
            pl.BlockSpec((1, 1), lambda i: (0, 0)),
            pl.BlockSpec((H, KGE), lambda i: (0, 0)),
            pl.BlockSpec((1, KGE), lambda i: (0, 0)),
        ],
        out_specs=pl.BlockSpec((G, KGE), lambda i: (0, 0)),
        out_shape=jax.ShapeDtypeStruct((G, KGE), jnp.float32),
    )(xc, batch2, wa1_t, ba1, aa, wa2, ba2, wr_t, br)


KD = 80
EPT = E // NW
N_PAD = 10240
NPT = N_PAD // NS
NCH = EPT // KD
NB2 = (NCH - 1) // 2

_MESH = plsc.VectorSubcoreMesh(core_axis_name="c", subcore_axis_name="s")


def _sc_nodeagg_body(fused2, dsti3, zrows, out,
                     didx_a, rows0, rows1, acc_sh,
                     lsem0, lsem1):
    cid = lax.axis_index("c")
    sid = lax.axis_index("s")
    wid = sid * NC + cid
    base = wid * EPT
    pltpu.sync_copy(zrows, acc_sh.at[pl.ds(sid * NPT, NPT)])
    pltpu.sync_copy(dsti3.at[wid], didx_a)
    plsc.subcore_barrier()

    def load(c, rows_v, sem):
        pltpu.async_copy(fused2.at[pl.ds(base + c * KD, KD)], rows_v, sem)

    def wait_load(rows_v, sem):
        pltpu.make_async_copy(fused2.at[pl.ds(base, KD)], rows_v, sem).wait()

    def scadd(c, rows_v):
        pltpu.sync_copy(rows_v, acc_sh.at[didx_a.at[c]], add=True)

    load(0, rows0, lsem0)

    def body(t, carry):
        c0 = 2 * t
        load(c0 + 1, rows1, lsem1)
        wait_load(rows0, lsem0)
        scadd(c0, rows0)
        load(c0 + 2, rows0, lsem0)
        wait_load(rows1, lsem1)
        scadd(c0 + 1, rows1)
        return carry

    lax.fori_loop(0, NB2, body, 0)
    wait_load(rows0, lsem0)
    scadd(NCH - 1, rows0)
    plsc.subcore_barrier()
    pltpu.sync_copy(acc_sh.at[pl.ds(sid * NPT, NPT)],
                    out.at[cid, pl.ds(sid * NPT, NPT)])


_sc_nodeagg = pl.kernel(
    _sc_nodeagg_body,
    out_type=jax.ShapeDtypeStruct((NC, N_PAD, H), jnp.float32),
    mesh=_MESH,
    scratch_types=[
        pltpu.VMEM((NCH, KD), jnp.int32),
        pltpu.VMEM((KD, H), jnp.float32),
        pltpu.VMEM((KD, H), jnp.float32),
        pltpu.VMEM_SHARED((N_PAD, H), jnp.float32),
        pltpu.SemaphoreType.DMA,
        pltpu.SemaphoreType.DMA,
    ],
)


def _sc_fuse_body(x, srci3, dsti3, ea, fused,
                  sidx_a, didx_a, xs0, xs1, xd0, xd1, ea0, ea1, out0, out1,
                  gsem0, gsem1, wsem0, wsem1):
    cid = lax.axis_index("c")
    sid = lax.axis_index("s")
    wid = sid * NC + cid
    base = wid * EPT
    pltpu.sync_copy(srci3.at[wid], sidx_a)
    pltpu.sync_copy(dsti3.at[wid], didx_a)

    def gathers(c, xs_v, xd_v, ea_v, sem):
        pltpu.async_copy(x.at[sidx_a.at[c]], xs_v, sem)
        pltpu.async_copy(x.at[didx_a.at[c]], xd_v, sem)
        pltpu.async_copy(ea.at[pl.ds(base + c * KD, KD)], ea_v, sem)

    def wait_gathers(xs_v, xd_v, ea_v, sem):
        pltpu.make_async_copy(x.at[sidx_a.at[0]], xs_v, sem).wait()
        pltpu.make_async_copy(x.at[didx_a.at[0]], xd_v, sem).wait()
        pltpu.make_async_copy(ea.at[pl.ds(base, KD)], ea_v, sem).wait()

    def compute(xs_v, xd_v, ea_v, out_v):
        def row(i, rcarry):
            for c8 in range(8):
                sl = pl.ds(c8 * 16, 16)
                out_v[i, sl] = (ea_v[i, sl] + 0.5 * xs_v[i, sl]
                                + 0.5 * xd_v[i, sl])
            return rcarry
        lax.fori_loop(0, KD, row, 0)

    gathers(0, xs0, xd0, ea0, gsem0)

    def body(t, carry):
        c0 = 2 * t
        gathers(c0 + 1, xs1, xd1, ea1, gsem1)
        wait_gathers(xs0, xd0, ea0, gsem0)
        compute(xs0, xd0, ea0, out0)
        pltpu.async_copy(out0, fused.at[pl.ds(base + c0 * KD, KD)], wsem0)
        gathers(c0 + 2, xs0, xd0, ea0, gsem0)
        wait_gathers(xs1, xd1, ea1, gsem1)
        compute(xs1, xd1, ea1, out1)
        pltpu.async_copy(out1, fused.at[pl.ds(base + (c0 + 1) * KD, KD)], wsem1)
        pltpu.make_async_copy(out0, fused.at[pl.ds(base, KD)], wsem0).wait()
        pltpu.make_async_copy(out1, fused.at[pl.ds(base, KD)], wsem1).wait()
        return carry

    lax.fori_loop(0, NB2, body, 0)
    wait_gathers(xs0, xd0, ea0, gsem0)
    compute(xs0, xd0, ea0, out0)
    pltpu.sync_copy(out0, fused.at[pl.ds(base + (NCH - 1) * KD, KD)])


_sc_fuse = pl.kernel(
    _sc_fuse_body,
    out_type=jax.ShapeDtypeStruct((E, H), jnp.float32),
    mesh=_MESH,
    scratch_types=[
        pltpu.VMEM((NCH, KD), jnp.int32),
        pltpu.VMEM((NCH, KD), jnp.int32),
        pltpu.VMEM((KD, H), jnp.float32),
        pltpu.VMEM((KD, H), jnp.float32),
        pltpu.VMEM((KD, H), jnp.float32),
        pltpu.VMEM((KD, H), jnp.float32),
        pltpu.VMEM((KD, H), jnp.float32),
        pltpu.VMEM((KD, H), jnp.float32),
        pltpu.VMEM((KD, H), jnp.float32),
        pltpu.VMEM((KD, H), jnp.float32),
        pltpu.SemaphoreType.DMA,
        pltpu.SemaphoreType.DMA,
        pltpu.SemaphoreType.DMA,
        pltpu.SemaphoreType.DMA,
    ],
)


def _sc_segsum_body(fused, gidx3, sidsa, aggbuf, prt, pidsf,
                    gidx_a, ids_a, oidx0, oidx1, rows0, rows1, stage0, stage1,
                    prow_v, pid_v, gsem0, gsem1, ssem0, ssem1):
    cid = lax.axis_index("c")
    sid = lax.axis_index("s")
    wid = sid * NC + cid
    base = wid * EPT
    iota16 = lax.iota(jnp.int32, 16)
    pltpu.sync_copy(gidx3.at[wid], gidx_a)
    pltpu.sync_copy(sidsa.at[pl.ds(base, EPT + 16)], ids_a)

    def compute(c, rows_v, stage_v, oidx_v, acc):
        coff = c * KD
        k0 = base + coff

        def group(g, acc):
            g0 = g * 16
            ids16 = ids_a[pl.ds(coff + g0, 16)]
            ids16n = ids_a[pl.ds(coff + g0 + 1, 16)]
            boundary = ids16 != ids16n
            keep16 = jnp.where(boundary, 0.0, 1.0)
            dump = (E + k0 + g0) + iota16
            oidx_v[pl.ds(g0, 16)] = jnp.where(boundary, ids16, dump)
            for j in range(16):
                r = g0 + j
                acc = tuple(acc[c8] + rows_v[r, pl.ds(c8 * 16, 16)]
                            for c8 in range(8))
                for c8 in range(8):
                    stage_v[r, pl.ds(c8 * 16, 16)] = acc[c8]
                kj = jnp.broadcast_to(lax.slice(keep16, (j,), (j + 1,)), (16,))
                acc = tuple(a * kj for a in acc)
            return acc

        return lax.fori_loop(0, KD // 16, group, acc)

    pltpu.async_copy(fused.at[gidx_a.at[0]], rows0, gsem0)

    def body(t, acc):
        c0 = 2 * t
        pltpu.async_copy(fused.at[gidx_a.at[c0 + 1]], rows1, gsem1)
        pltpu.make_async_copy(fused.at[gidx_a.at[0]], rows0, gsem0).wait()
        acc = compute(c0, rows0, stage0, oidx0, acc)
        pltpu.async_copy(stage0, aggbuf.at[oidx0], ssem0)
        pltpu.async_copy(fused.at[gidx_a.at[c0 + 2]], rows0, gsem0)
        pltpu.make_async_copy(fused.at[gidx_a.at[0]], rows1, gsem1).wait()
        acc = compute(c0 + 1, rows1, stage1, oidx1, acc)
        pltpu.async_copy(stage1, aggbuf.at[oidx1], ssem1)
        pltpu.make_async_copy(stage0, aggbuf.at[oidx0], ssem0).wait()
        pltpu.make_async_copy(stage1, aggbuf.at[oidx1], ssem1).wait()
        return acc

    zero16 = jnp.zeros((16,), jnp.float32)
    acc = lax.fori_loop(0, NB2, body, (zero16,) * 8)
    pltpu.make_async_copy(fused.at[gidx_a.at[0]], rows0, gsem0).wait()
    acc = compute(NCH - 1, rows0, stage0, oidx0, acc)
    pltpu.sync_copy(stage0, aggbuf.at[oidx0])
    for c8 in range(8):
        prow_v[pl.ds(c8 * 16, 16)] = acc[c8]
    pltpu.sync_copy(prow_v, prt.at[pl.ds(wid * H, H)])
    pltpu.sync_copy(sidsa.at[pl.ds(base + EPT - 16, 16)], pid_v)
    pltpu.sync_copy(pid_v, pidsf.at[pl.ds(wid * L, L)])


_sc_segsum = pl.kernel(
    _sc_segsum_body,
    out_type=(jax.ShapeDtypeStruct((2 * E, H), jnp.float32),
              jax.ShapeDtypeStruct((NW * H,), jnp.float32),
              jax.ShapeDtypeStruct((NW * L,), jnp.int32)),
    mesh=_MESH,
    scratch_types=[
        pltpu.VMEM((NCH, KD), jnp.int32),
        pltpu.VMEM((EPT + 16,), jnp.int32),
        pltpu.VMEM((KD,), jnp.int32),
        pltpu.VMEM((KD,), jnp.int32),
        pltpu.VMEM((KD, H), jnp.float32),
        pltpu.VMEM((KD, H), jnp.float32),
        pltpu.VMEM((KD, H), jnp.float32),
        pltpu.VMEM((KD, H), jnp.float32),
        pltpu.VMEM((H,), jnp.float32),
        pltpu.VMEM((L,), jnp.int32),
        pltpu.SemaphoreType.DMA,
        pltpu.SemaphoreType.DMA,
        pltpu.SemaphoreType.DMA,
        pltpu.SemaphoreType.DMA,
    ],
)


def kernel(x, edge_index, edge_attr, batch, line_graph_edge_index,
           W_e, b_e, a_e, gamma_bn, beta_bn,
           W_ih, W_hh, b_ih, b_hh,
           W_a1, b_a1, a_a, W_a2, b_a2, W_r, b_r):
    src = edge_index[0]
    dst = edge_index[1]
    lsrc = line_graph_edge_index[0]
    ldst = line_graph_edge_index[1]

    perm = jnp.argsort(ldst)
    gidx3 = lsrc[perm].reshape(NW, NCH, KD)
    sids_pad = jnp.concatenate(
        [ldst[perm], jnp.full((16,), -1, jnp.int32)])
    src3 = src.reshape(NW, NCH, KD)
    dst3 = dst.reshape(NW, NCH, KD)
    cnt_l = jax.ops.segment_sum(jnp.ones((E, 1), jnp.float32),
                                sids_pad[:E], num_segments=E,
                                indices_are_sorted=True)
    cnt_d = jax.ops.segment_sum(jnp.ones((E, 1), jnp.float32), dst,
                                num_segments=N)

    w_e_t = W_e.T
    b_e2 = b_e.reshape(1, H)
    gamma2 = gamma_bn.reshape(1, H)
    beta2 = beta_bn.reshape(1, H)
    wih_t = W_ih.T
    whh_t = W_hh.T
    bih2 = b_ih.reshape(1, 3 * H)
    bhh2 = b_hh.reshape(1, 3 * H)
    a_e1 = a_e.reshape(1)
    a_a1 = a_a.reshape(1)
    batch2 = batch.reshape(N, 1)

    pids0 = jnp.full((NW, L), -1, jnp.int32)
    P0 = jnp.zeros((NW, H), jnp.float32)
    zrows = jnp.zeros((NPT, H), jnp.float32)

    hidden = x
    xc = x
    for _ in range(N_ITER):
        fused = _sc_fuse(xc, src3, dst3, edge_attr)
        aggbuf, prt, pidsf = _sc_segsum(fused, gidx3, sids_pad)
        pids = pidsf.reshape(NW, L)
        P = prt.reshape(NW, H)
        stats = _tc_stats(aggbuf, pids, P, cnt_l, w_e_t, b_e2, a_e1)
        fused2 = _tc_fuse2(aggbuf, pids, P, cnt_l, fused, stats, w_e_t, b_e2,
                           a_e1, gamma2, beta2)
        nu2 = _sc_nodeagg(fused2, dst3, zrows)
        hidden = _tc_gru(nu2, cnt_d, hidden, wih_t, whh_t, bih2, bhh2)
        xc = hidden

    graph_emb = _tc_readout(xc, batch2, W_a1.T, b_a1.reshape(1, H // 2), a_a1,
                            W_a2.reshape(1, H // 2), b_a2.reshape(1, 1),
                            W_r.T, b_r.reshape(1, KGE))
    return (xc, graph_emb)
